# baseline (speedup 1.0000x reference)
import jax
import jax.numpy as jnp
from jax.experimental import pallas as pl
from jax.experimental.pallas import tpu as pltpu

_BIG = 2**30
_K = 100


def _pick_chunk(n):
    for r in (80, 160, 40, 8, 1):
        if n % r == 0:
            return r
    return 1


def _make_body(N, C, R, T, K):
    neg_inf = float("-inf")

    def body(x_ref, bx_ref, os_ref, ol_ref, ob_ref, cmv_ref):
        def init_t(t, _):
            blk = x_ref[0, pl.ds(t * R, R), :]
            cmv_ref[pl.ds(t, 1), :] = jnp.max(blk, axis=0, keepdims=True)
            return 0

        jax.lax.fori_loop(0, T, init_t, 0)

        rowi = jax.lax.broadcasted_iota(jnp.int32, (R, C), 0)
        coli = jax.lax.broadcasted_iota(jnp.int32, (R, C), 1)
        flat0 = rowi * C + coli
        ti = jax.lax.broadcasted_iota(jnp.int32, (T, C), 0)

        def extract(k, _):
            cv = cmv_ref[...]
            m = jnp.max(cv)
            t = jnp.min(jnp.where(cv == m, ti, _BIG))

            blk = x_ref[0, pl.ds(t * R, R), :]
            cand = jnp.where(blk == m, t * (R * C) + flat0, _BIG)
            fi = jnp.min(cand)
            row = fi // C
            col = fi - row * C

            os_ref[0, pl.ds(k, 1), :] = m[None, None]
            ol_ref[0, pl.ds(k, 1), :] = col[None, None]
            ob_ref[0, pl.ds(k, 1), :] = bx_ref[0, pl.ds(row, 1), :]

            new_blk = jnp.where(t * (R * C) + flat0 == fi, neg_inf, blk)
            x_ref[0, pl.ds(t * R, R), :] = new_blk
            cmv_ref[pl.ds(t, 1), :] = jnp.max(new_blk, axis=0, keepdims=True)
            return 0

        jax.lax.fori_loop(0, K, extract, 0)

    return body


def kernel(pred_scores, pred_boxes):
    B, N, C = pred_scores.shape
    R = _pick_chunk(N)
    T = N // R
    K = _K

    s3, l3, b3 = pl.pallas_call(
        _make_body(N, C, R, T, K),
        grid=(B,),
        in_specs=[
            pl.BlockSpec((1, N, C), lambda b: (b, 0, 0)),
            pl.BlockSpec((1, N, 4), lambda b: (b, 0, 0)),
        ],
        out_specs=[
            pl.BlockSpec((1, K, 1), lambda b: (b, 0, 0)),
            pl.BlockSpec((1, K, 1), lambda b: (b, 0, 0)),
            pl.BlockSpec((1, K, 4), lambda b: (b, 0, 0)),
        ],
        out_shape=[
            jax.ShapeDtypeStruct((B, K, 1), jnp.float32),
            jax.ShapeDtypeStruct((B, K, 1), jnp.int32),
            jax.ShapeDtypeStruct((B, K, 4), jnp.float32),
        ],
        scratch_shapes=[
            pltpu.VMEM((T, C), jnp.float32),
        ],
    )(pred_scores, pred_boxes)

    return s3[:, :, 0], l3[:, :, 0], b3


# speedup vs baseline: 1.0039x; 1.0039x over previous
import jax
import jax.numpy as jnp
from jax.experimental import pallas as pl
from jax.experimental.pallas import tpu as pltpu

_BIG = 2**30
_K = 100


def _pick_chunk(n):
    for r in (160, 80, 40, 8, 1):
        if n % r == 0:
            return r
    return 1


def _make_body(N, C, R, T, K):
    neg_inf = float("-inf")

    def body(x_ref, bx_ref, os_ref, ol_ref, ob_ref, cmv_ref):
        def init_t(t, _):
            blk = x_ref[0, pl.ds(t * R, R), :]
            cmv_ref[pl.ds(t, 1), :] = jnp.max(blk, axis=0, keepdims=True)
            return 0

        jax.lax.fori_loop(0, T, init_t, 0)

        rowi = jax.lax.broadcasted_iota(jnp.int32, (R, C), 0)
        coli = jax.lax.broadcasted_iota(jnp.int32, (R, C), 1)
        flat0 = rowi * C + coli
        ti = jax.lax.broadcasted_iota(jnp.int32, (T, C), 0)

        def extract(k, _):
            cv = cmv_ref[...]
            m = jnp.max(cv)
            t = jnp.min(jnp.where(cv == m, ti, _BIG))

            blk = x_ref[0, pl.ds(t * R, R), :]
            cand = jnp.where(blk == m, t * (R * C) + flat0, _BIG)
            fi = jnp.min(cand)
            row = fi // C
            col = fi - row * C

            os_ref[0, pl.ds(k, 1), :] = m[None, None]
            ol_ref[0, pl.ds(k, 1), :] = col[None, None]
            ob_ref[0, pl.ds(k, 1), :] = bx_ref[0, pl.ds(row, 1), :]

            new_blk = jnp.where(t * (R * C) + flat0 == fi, neg_inf, blk)
            x_ref[0, pl.ds(t * R, R), :] = new_blk
            cmv_ref[pl.ds(t, 1), :] = jnp.max(new_blk, axis=0, keepdims=True)
            return 0

        jax.lax.fori_loop(0, K, extract, 0)

    return body


def kernel(pred_scores, pred_boxes):
    B, N, C = pred_scores.shape
    R = _pick_chunk(N)
    T = N // R
    K = _K

    s3, l3, b3 = pl.pallas_call(
        _make_body(N, C, R, T, K),
        grid=(B,),
        in_specs=[
            pl.BlockSpec((1, N, C), lambda b: (b, 0, 0)),
            pl.BlockSpec((1, N, 4), lambda b: (b, 0, 0)),
        ],
        out_specs=[
            pl.BlockSpec((1, K, 1), lambda b: (b, 0, 0)),
            pl.BlockSpec((1, K, 1), lambda b: (b, 0, 0)),
            pl.BlockSpec((1, K, 4), lambda b: (b, 0, 0)),
        ],
        out_shape=[
            jax.ShapeDtypeStruct((B, K, 1), jnp.float32),
            jax.ShapeDtypeStruct((B, K, 1), jnp.int32),
            jax.ShapeDtypeStruct((B, K, 4), jnp.float32),
        ],
        scratch_shapes=[
            pltpu.VMEM((T, C), jnp.float32),
        ],
    )(pred_scores, pred_boxes)

    return s3[:, :, 0], l3[:, :, 0], b3


# speedup vs baseline: 1.2766x; 1.2716x over previous
import jax
import jax.numpy as jnp
from jax.experimental import pallas as pl
from jax.experimental.pallas import tpu as pltpu

_BIG = 2**30
_K = 100


def _pick_chunk(n):
    for r in (160, 80, 40, 8, 1):
        if n % r == 0:
            return r
    return 1


def _make_body(N, C, R, T, K):
    neg_inf = float("-inf")

    def body(x_ref, bx_ref, os_ref, ol_ref, ob_ref, cmv_ref, cmi_ref):
        rowi = jax.lax.broadcasted_iota(jnp.int32, (R, C), 0)
        coli = jax.lax.broadcasted_iota(jnp.int32, (R, C), 1)
        flat0 = rowi * C + coli

        def chunk_stats(t):
            blk = x_ref[0, pl.ds(t * R, R), :]
            bm = jnp.max(blk, axis=0, keepdims=True)
            cand = jnp.where(blk == bm, t * (R * C) + flat0, _BIG)
            bi = jnp.min(cand, axis=0, keepdims=True)
            return bm, bi

        def init_t(t, _):
            bm, bi = chunk_stats(t)
            cmv_ref[pl.ds(t, 1), :] = bm
            cmi_ref[pl.ds(t, 1), :] = bi
            return 0

        jax.lax.fori_loop(0, T, init_t, 0)

        def extract(k, _):
            cv = cmv_ref[...]
            m = jnp.max(cv)
            fi = jnp.min(jnp.where(cv == m, cmi_ref[...], _BIG))
            row = fi // C
            col = fi - row * C
            t = row // R

            os_ref[0, pl.ds(k, 1), :] = m[None, None]
            ol_ref[0, pl.ds(k, 1), :] = col[None, None]
            ob_ref[0, pl.ds(k, 1), :] = bx_ref[0, pl.ds(row, 1), :]

            rowv = x_ref[0, pl.ds(row, 1), :]
            li = jax.lax.broadcasted_iota(jnp.int32, (1, C), 1)
            x_ref[0, pl.ds(row, 1), :] = jnp.where(li == col, neg_inf, rowv)

            bm, bi = chunk_stats(t)
            cmv_ref[pl.ds(t, 1), :] = bm
            cmi_ref[pl.ds(t, 1), :] = bi
            return 0

        jax.lax.fori_loop(0, K, extract, 0)

    return body


def kernel(pred_scores, pred_boxes):
    B, N, C = pred_scores.shape
    R = _pick_chunk(N)
    T = N // R
    K = _K

    s3, l3, b3 = pl.pallas_call(
        _make_body(N, C, R, T, K),
        grid=(B,),
        in_specs=[
            pl.BlockSpec((1, N, C), lambda b: (b, 0, 0)),
            pl.BlockSpec((1, N, 4), lambda b: (b, 0, 0)),
        ],
        out_specs=[
            pl.BlockSpec((1, K, 1), lambda b: (b, 0, 0)),
            pl.BlockSpec((1, K, 1), lambda b: (b, 0, 0)),
            pl.BlockSpec((1, K, 4), lambda b: (b, 0, 0)),
        ],
        out_shape=[
            jax.ShapeDtypeStruct((B, K, 1), jnp.float32),
            jax.ShapeDtypeStruct((B, K, 1), jnp.int32),
            jax.ShapeDtypeStruct((B, K, 4), jnp.float32),
        ],
        scratch_shapes=[
            pltpu.VMEM((T, C), jnp.float32),
            pltpu.VMEM((T, C), jnp.int32),
        ],
    )(pred_scores, pred_boxes)

    return s3[:, :, 0], l3[:, :, 0], b3


# speedup vs baseline: 1.3076x; 1.0243x over previous
import jax
import jax.numpy as jnp
from jax.experimental import pallas as pl
from jax.experimental.pallas import tpu as pltpu

_BIG = 2**30
_K = 100


def _pick_chunk(n):
    for r in (160, 80, 40, 8, 1):
        if n % r == 0:
            return r
    return 1


def _make_body(N, C, R, T, K, P):
    neg_inf = float("-inf")

    def body(x_ref, bx_hbm, os_ref, ol_ref, ob_ref, cmv_ref, cmi_ref, sem):
        rowi = jax.lax.broadcasted_iota(jnp.int32, (R, C), 0)
        coli = jax.lax.broadcasted_iota(jnp.int32, (R, C), 1)
        flat0 = rowi * C + coli
        b0 = pl.program_id(0) * P

        def chunk_stats(s, t):
            blk = x_ref[s, pl.ds(t * R, R), :]
            bm = jnp.max(blk, axis=0, keepdims=True)
            cand = jnp.where(blk == bm, t * (R * C) + flat0, _BIG)
            bi = jnp.min(cand, axis=0, keepdims=True)
            return bm, bi

        def init_t(t, _):
            for s in range(P):
                bm, bi = chunk_stats(s, t)
                cmv_ref[s, pl.ds(t, 1), :] = bm
                cmi_ref[s, pl.ds(t, 1), :] = bi
            return 0

        jax.lax.fori_loop(0, T, init_t, 0)

        def extract(k, _):
            for s in range(P):
                cv = cmv_ref[s]
                m = jnp.max(cv)
                fi = jnp.min(jnp.where(cv == m, cmi_ref[s], _BIG))
                row = fi // C
                col = fi - row * C
                t = row // R

                os_ref[s, pl.ds(k, 1), :] = m[None, None]
                ol_ref[s, pl.ds(k, 1), :] = col[None, None]
                pltpu.make_async_copy(
                    bx_hbm.at[pl.ds(b0 + s, 1), pl.ds(row, 1), :],
                    ob_ref.at[pl.ds(s, 1), pl.ds(k, 1), :],
                    sem,
                ).start()

                rowv = x_ref[s, pl.ds(row, 1), :]
                li = jax.lax.broadcasted_iota(jnp.int32, (1, C), 1)
                x_ref[s, pl.ds(row, 1), :] = jnp.where(li == col, neg_inf, rowv)

                bm, bi = chunk_stats(s, t)
                cmv_ref[s, pl.ds(t, 1), :] = bm
                cmi_ref[s, pl.ds(t, 1), :] = bi
            return 0

        jax.lax.fori_loop(0, K, extract, 0)

        drain = pltpu.make_async_copy(
            bx_hbm.at[pl.ds(0, 1), pl.ds(0, 1), :],
            ob_ref.at[pl.ds(0, 1), pl.ds(0, 1), :],
            sem,
        )
        for _ in range(P * K):
            drain.wait()

    return body


def kernel(pred_scores, pred_boxes):
    B, N, C = pred_scores.shape
    R = _pick_chunk(N)
    T = N // R
    K = _K
    P = 2 if B % 2 == 0 else 1

    s3, l3, b3 = pl.pallas_call(
        _make_body(N, C, R, T, K, P),
        grid=(B // P,),
        in_specs=[
            pl.BlockSpec((P, N, C), lambda b: (b, 0, 0)),
            pl.BlockSpec(memory_space=pl.ANY),
        ],
        out_specs=[
            pl.BlockSpec((P, K, 1), lambda b: (b, 0, 0)),
            pl.BlockSpec((P, K, 1), lambda b: (b, 0, 0)),
            pl.BlockSpec((P, K, 4), lambda b: (b, 0, 0)),
        ],
        out_shape=[
            jax.ShapeDtypeStruct((B, K, 1), jnp.float32),
            jax.ShapeDtypeStruct((B, K, 1), jnp.int32),
            jax.ShapeDtypeStruct((B, K, 4), jnp.float32),
        ],
        scratch_shapes=[
            pltpu.VMEM((P, T, C), jnp.float32),
            pltpu.VMEM((P, T, C), jnp.int32),
            pltpu.SemaphoreType.DMA,
        ],
    )(pred_scores, pred_boxes)

    return s3[:, :, 0], l3[:, :, 0], b3
